# Initial kernel scaffold; baseline (speedup 1.0000x reference)
import jax
import jax.numpy as jnp
from jax.experimental import pallas as pl


def kernel(h, x, edge_index, edge_attr, We1, be1, We2, be2, Wc1, bc1, Wc2, Wn1, bn1, Wn2, bn2):
    raise NotImplementedError("write your pallas kernel here")


# speedup vs baseline: 2.2274x; 2.2274x over previous
import functools

import jax
import jax.numpy as jnp
from jax import lax
from jax.experimental import pallas as pl
from jax.experimental.pallas import tpu as pltpu
from jax.experimental.pallas import tpu_sc as plsc

N = 10000
E = 320000
D = 128
NC = 2
NS = 16
NW = NC * NS
EW = E // NW
C = 80
NCH = EW // C
NP = 10112
RROWS = NP // NS
XS = 10240

BN = 1000
BE = 512

_HI = lax.Precision.HIGHEST

_sc_mesh = plsc.VectorSubcoreMesh(core_axis_name="c", subcore_axis_name="s")


def _pre_body(h_ref, wa_ref, wb_ref, be1_ref, t1_ref, t2_ref):
    hb = h_ref[...]
    t1_ref[...] = jnp.dot(hb, wa_ref[...], precision=_HI) + be1_ref[...]
    t2_ref[...] = jnp.dot(hb, wb_ref[...], precision=_HI)


def _node_pre(h, wa, wb, be1):
    return pl.pallas_call(
        _pre_body,
        grid=(N // BN,),
        in_specs=[
            pl.BlockSpec((BN, D), lambda i: (i, 0)),
            pl.BlockSpec((D, D), lambda i: (0, 0)),
            pl.BlockSpec((D, D), lambda i: (0, 0)),
            pl.BlockSpec((1, D), lambda i: (0, 0)),
        ],
        out_specs=[
            pl.BlockSpec((BN, D), lambda i: (i, 0)),
            pl.BlockSpec((BN, D), lambda i: (i, 0)),
        ],
        out_shape=[
            jax.ShapeDtypeStruct((N, D), jnp.float32),
            jax.ShapeDtypeStruct((N, D), jnp.float32),
        ],
    )(h, wa, wb, be1)


@functools.partial(
    pl.kernel,
    out_type=(
        jax.ShapeDtypeStruct((E, D), jnp.float32),
        jax.ShapeDtypeStruct((E, D), jnp.float32),
        jax.ShapeDtypeStruct((E,), jnp.float32),
        jax.ShapeDtypeStruct((E,), jnp.float32),
        jax.ShapeDtypeStruct((E,), jnp.float32),
    ),
    mesh=_sc_mesh,
    compiler_params=pltpu.CompilerParams(needs_layout_passes=False),
    scratch_types=[
        pltpu.VMEM((C,), jnp.int32),
        pltpu.VMEM((C,), jnp.int32),
        pltpu.VMEM((C, D), jnp.float32),
        pltpu.VMEM((C, D), jnp.float32),
        pltpu.VMEM((3 * XS,), jnp.float32),
        pltpu.VMEM((C,), jnp.float32),
        pltpu.VMEM((C,), jnp.float32),
        pltpu.VMEM((C,), jnp.float32),
        pltpu.SemaphoreType.DMA,
        pltpu.SemaphoreType.DMA,
    ],
)
def _sc_gather(t1_hbm, t2_hbm, xf_hbm, row_hbm, col_hbm,
               br_hbm, bc_hbm, d0_hbm, d1_hbm, d2_hbm,
               idxr, idxc, b1, b2, xt, bd0, bd1, bd2, s1, s2):
    wid = lax.axis_index("s") * NC + lax.axis_index("c")
    base0 = wid * EW

    pltpu.sync_copy(xf_hbm, xt)

    @pl.loop(0, NCH)
    def _chunk(j):
        base = base0 + j * C
        pltpu.sync_copy(row_hbm.at[pl.ds(base, C)], idxr)
        pltpu.sync_copy(col_hbm.at[pl.ds(base, C)], idxc)
        cp1 = pltpu.async_copy(t1_hbm.at[idxr], b1, s1)
        cp2 = pltpu.async_copy(t2_hbm.at[idxc], b2, s2)

        @pl.loop(0, C // 16)
        def _grp(k):
            sl = pl.ds(k * 16, 16)
            ir = idxr[sl]
            ic = idxc[sl]
            bd0[sl] = plsc.load_gather(xt, [ir]) - plsc.load_gather(xt, [ic])
            ir1 = ir + XS
            ic1 = ic + XS
            bd1[sl] = plsc.load_gather(xt, [ir1]) - plsc.load_gather(xt, [ic1])
            ir2 = ir1 + XS
            ic2 = ic1 + XS
            bd2[sl] = plsc.load_gather(xt, [ir2]) - plsc.load_gather(xt, [ic2])

        cp1.wait()
        cp2.wait()
        pltpu.sync_copy(b1, br_hbm.at[pl.ds(base, C)])
        pltpu.sync_copy(b2, bc_hbm.at[pl.ds(base, C)])
        pltpu.sync_copy(bd0, d0_hbm.at[pl.ds(base, C)])
        pltpu.sync_copy(bd1, d1_hbm.at[pl.ds(base, C)])
        pltpu.sync_copy(bd2, d2_hbm.at[pl.ds(base, C)])


def _edge_body(br_ref, bc_ref, d0_ref, d1_ref, d2_ref, ea_ref,
               b3_ref, wd_ref, we_ref, we2_ref, be2_ref,
               wc1_ref, bc1_ref, wc2_ref,
               m_ref, t_ref):
    d0r = d0_ref[...].reshape(1, BE)
    d1r = d1_ref[...].reshape(1, BE)
    d2r = d2_ref[...].reshape(1, BE)
    dr3 = jnp.concatenate([d0r, d1r, d2r], axis=0)
    d16 = lax.dot_general(dr3, b3_ref[...], (((0,), (0,)), ((), ())),
                          precision=_HI)
    ds = jnp.sum(d16 * d16, axis=1, keepdims=True)
    pre = (br_ref[...] + bc_ref[...] + ds * wd_ref[...]
           + jnp.dot(ea_ref[...], we_ref[...], precision=_HI))
    m1 = jax.nn.silu(pre)
    m2 = jax.nn.silu(jnp.dot(m1, we2_ref[...], precision=_HI) + be2_ref[...])
    mc = jax.nn.silu(jnp.dot(m2, wc1_ref[...], precision=_HI) + bc1_ref[...])
    w = lax.dot_general(mc, wc2_ref[...], (((1,), (1,)), ((), ())),
                        precision=_HI)
    s = jnp.tanh(w) / (jnp.sqrt(ds + 1e-08) + 1e-08)
    m_ref[...] = m2
    t_ref[...] = jnp.concatenate(
        [d16 * s, jnp.zeros((BE, D - 16), jnp.float32)], axis=1)


def _edge_mlp(br, bc, d0, d1, d2, ea, b3, wd, we, we2, be2, wc1, bc1, wc2):
    return pl.pallas_call(
        _edge_body,
        grid=(E // BE,),
        in_specs=[
            pl.BlockSpec((BE, D), lambda i: (i, 0)),
            pl.BlockSpec((BE, D), lambda i: (i, 0)),
            pl.BlockSpec((BE,), lambda i: (i,)),
            pl.BlockSpec((BE,), lambda i: (i,)),
            pl.BlockSpec((BE,), lambda i: (i,)),
            pl.BlockSpec((BE, 16), lambda i: (i, 0)),
            pl.BlockSpec((3, 16), lambda i: (0, 0)),
            pl.BlockSpec((1, D), lambda i: (0, 0)),
            pl.BlockSpec((16, D), lambda i: (0, 0)),
            pl.BlockSpec((D, D), lambda i: (0, 0)),
            pl.BlockSpec((1, D), lambda i: (0, 0)),
            pl.BlockSpec((D, D), lambda i: (0, 0)),
            pl.BlockSpec((1, D), lambda i: (0, 0)),
            pl.BlockSpec((1, D), lambda i: (0, 0)),
        ],
        out_specs=[
            pl.BlockSpec((BE, D), lambda i: (i, 0)),
            pl.BlockSpec((BE, D), lambda i: (i, 0)),
        ],
        out_shape=[
            jax.ShapeDtypeStruct((E, D), jnp.float32),
            jax.ShapeDtypeStruct((E, D), jnp.float32),
        ],
    )(br, bc, d0, d1, d2, ea, b3, wd, we, we2, be2, wc1, bc1, wc2)


@functools.partial(
    pl.kernel,
    out_type=(
        jax.ShapeDtypeStruct((NC, NP, D), jnp.float32),
        jax.ShapeDtypeStruct((NC, NP, D), jnp.float32),
    ),
    mesh=_sc_mesh,
    compiler_params=pltpu.CompilerParams(needs_layout_passes=False),
    scratch_types=[
        pltpu.VMEM((C,), jnp.int32),
        pltpu.VMEM((C, D), jnp.float32),
        pltpu.VMEM_SHARED((NP, D), jnp.float32),
    ],
)
def _sc_scatter(m_hbm, t_hbm, row_hbm, zeros_hbm, pm_hbm, pt_hbm,
                idx, buf, acc):
    c = lax.axis_index("c")
    s = lax.axis_index("s")
    slab = pl.ds(s * RROWS, RROWS)
    base0 = (s * NC + c) * EW

    pltpu.sync_copy(zeros_hbm.at[slab], acc.at[slab])
    plsc.subcore_barrier()

    @pl.loop(0, NCH)
    def _mchunk(j):
        base = base0 + j * C
        pltpu.sync_copy(row_hbm.at[pl.ds(base, C)], idx)
        pltpu.sync_copy(m_hbm.at[pl.ds(base, C)], buf)
        pltpu.sync_copy(buf, acc.at[idx], add=True)

    plsc.subcore_barrier()
    pltpu.sync_copy(acc.at[slab], pm_hbm.at[c, slab])
    pltpu.sync_copy(zeros_hbm.at[slab], acc.at[slab])
    plsc.subcore_barrier()

    @pl.loop(0, NCH)
    def _tchunk(j):
        base = base0 + j * C
        pltpu.sync_copy(row_hbm.at[pl.ds(base, C)], idx)
        pltpu.sync_copy(t_hbm.at[pl.ds(base, C)], buf)
        pltpu.sync_copy(buf, acc.at[idx], add=True)

    plsc.subcore_barrier()
    pltpu.sync_copy(acc.at[slab], pt_hbm.at[c, slab])


def _node_body(h_ref, xp_ref, pm0_ref, pm1_ref, pt0_ref, pt1_ref,
               wn1h_ref, wn1m_ref, bn1_ref, wn2_ref, bn2_ref,
               hn_ref, xn_ref):
    hb = h_ref[...]
    aggm = pm0_ref[...] + pm1_ref[...]
    aggt = pt0_ref[...] + pt1_ref[...]
    u = jax.nn.silu(jnp.dot(hb, wn1h_ref[...], precision=_HI)
                    + jnp.dot(aggm, wn1m_ref[...], precision=_HI)
                    + bn1_ref[...])
    hn_ref[...] = hb + jnp.dot(u, wn2_ref[...], precision=_HI) + bn2_ref[...]
    xn_ref[...] = xp_ref[...] + aggt[:, :16]


def _node_mlp(h, xp, pm0, pm1, pt0, pt1, wn1h, wn1m, bn1, wn2, bn2):
    return pl.pallas_call(
        _node_body,
        grid=(N // BN,),
        in_specs=[
            pl.BlockSpec((BN, D), lambda i: (i, 0)),
            pl.BlockSpec((BN, 16), lambda i: (i, 0)),
            pl.BlockSpec((BN, D), lambda i: (i, 0)),
            pl.BlockSpec((BN, D), lambda i: (i, 0)),
            pl.BlockSpec((BN, D), lambda i: (i, 0)),
            pl.BlockSpec((BN, D), lambda i: (i, 0)),
            pl.BlockSpec((D, D), lambda i: (0, 0)),
            pl.BlockSpec((D, D), lambda i: (0, 0)),
            pl.BlockSpec((1, D), lambda i: (0, 0)),
            pl.BlockSpec((D, D), lambda i: (0, 0)),
            pl.BlockSpec((1, D), lambda i: (0, 0)),
        ],
        out_specs=[
            pl.BlockSpec((BN, D), lambda i: (i, 0)),
            pl.BlockSpec((BN, 16), lambda i: (i, 0)),
        ],
        out_shape=[
            jax.ShapeDtypeStruct((N, D), jnp.float32),
            jax.ShapeDtypeStruct((N, 16), jnp.float32),
        ],
    )(h, xp, pm0, pm1, pt0, pt1, wn1h, wn1m, bn1, wn2, bn2)


def kernel(h, x, edge_index, edge_attr,
           We1, be1, We2, be2, Wc1, bc1, Wc2, Wn1, bn1, Wn2, bn2):
    row = edge_index[0].astype(jnp.int32)
    col = edge_index[1].astype(jnp.int32)
    xf = jnp.pad(x.T, ((0, 0), (0, XS - N))).reshape(-1)
    xp = jnp.pad(x, ((0, 0), (0, 13)))

    wa = We1[:D]
    wb = We1[D:2 * D]
    wd = We1[2 * D:2 * D + 1]
    we = We1[2 * D + 1:]
    b3 = jnp.eye(3, 16, dtype=jnp.float32)

    t1, t2 = _node_pre(h, wa, wb, be1.reshape(1, D))
    br, bc, d0, d1, d2 = _sc_gather(t1, t2, xf, row, col)
    m_ij, tt = _edge_mlp(br, bc, d0, d1, d2, edge_attr, b3,
                         wd, we, We2, be2.reshape(1, D),
                         Wc1, bc1.reshape(1, D), Wc2.reshape(1, D))

    zeros = jnp.zeros((NP, D), jnp.float32)
    pm, pt = _sc_scatter(m_ij, tt, row, zeros)
    pm = pm[:, :N]
    pt = pt[:, :N]

    hn, xn = _node_mlp(h, xp, pm[0], pm[1], pt[0], pt[1],
                       Wn1[:D], Wn1[D:], bn1.reshape(1, D),
                       Wn2, bn2.reshape(1, D))
    return hn, xn[:, :3]


# speedup vs baseline: 2.7069x; 1.2153x over previous
import functools

import jax
import jax.numpy as jnp
from jax import lax
from jax.experimental import pallas as pl
from jax.experimental.pallas import tpu as pltpu
from jax.experimental.pallas import tpu_sc as plsc

N = 10000
E = 320000
D = 128
NC = 2
NS = 16
NW = NC * NS
EW = E // NW
C = 80
NCH = EW // C
NP = 10112
RROWS = NP // NS
XS = 10240

BN = 1000
BE = 512

_HI = lax.Precision.HIGHEST

_sc_mesh = plsc.VectorSubcoreMesh(core_axis_name="c", subcore_axis_name="s")


def _pre_body(h_ref, wa_ref, wb_ref, be1_ref, t1_ref, t2_ref):
    hb = h_ref[...]
    t1_ref[...] = jnp.dot(hb, wa_ref[...], precision=_HI) + be1_ref[...]
    t2_ref[...] = jnp.dot(hb, wb_ref[...], precision=_HI)


def _node_pre(h, wa, wb, be1):
    return pl.pallas_call(
        _pre_body,
        grid=(N // BN,),
        in_specs=[
            pl.BlockSpec((BN, D), lambda i: (i, 0)),
            pl.BlockSpec((D, D), lambda i: (0, 0)),
            pl.BlockSpec((D, D), lambda i: (0, 0)),
            pl.BlockSpec((1, D), lambda i: (0, 0)),
        ],
        out_specs=[
            pl.BlockSpec((BN, D), lambda i: (i, 0)),
            pl.BlockSpec((BN, D), lambda i: (i, 0)),
        ],
        out_shape=[
            jax.ShapeDtypeStruct((N, D), jnp.float32),
            jax.ShapeDtypeStruct((N, D), jnp.float32),
        ],
    )(h, wa, wb, be1)


@functools.partial(
    pl.kernel,
    out_type=(
        jax.ShapeDtypeStruct((E, D), jnp.float32),
        jax.ShapeDtypeStruct((E, D), jnp.float32),
        jax.ShapeDtypeStruct((E,), jnp.float32),
        jax.ShapeDtypeStruct((E,), jnp.float32),
        jax.ShapeDtypeStruct((E,), jnp.float32),
    ),
    mesh=_sc_mesh,
    compiler_params=pltpu.CompilerParams(needs_layout_passes=False),
    scratch_types=[
        pltpu.VMEM((C,), jnp.int32),
        pltpu.VMEM((C,), jnp.int32),
        pltpu.VMEM((C, D), jnp.float32),
        pltpu.VMEM((C, D), jnp.float32),
        pltpu.VMEM((3 * XS,), jnp.float32),
        pltpu.VMEM((C,), jnp.float32),
        pltpu.VMEM((C,), jnp.float32),
        pltpu.VMEM((C,), jnp.float32),
        pltpu.SemaphoreType.DMA,
        pltpu.SemaphoreType.DMA,
    ],
)
def _sc_gather(t1_hbm, t2_hbm, xf_hbm, row_hbm, col_hbm,
               br_hbm, bc_hbm, d0_hbm, d1_hbm, d2_hbm,
               idxr, idxc, b1, b2, xt, bd0, bd1, bd2, s1, s2):
    wid = lax.axis_index("s") * NC + lax.axis_index("c")
    base0 = wid * EW

    pltpu.sync_copy(xf_hbm, xt)

    @pl.loop(0, NCH)
    def _chunk(j):
        base = base0 + j * C
        pltpu.sync_copy(row_hbm.at[pl.ds(base, C)], idxr)
        pltpu.sync_copy(col_hbm.at[pl.ds(base, C)], idxc)
        cp1 = pltpu.async_copy(t1_hbm.at[idxr], b1, s1)
        cp2 = pltpu.async_copy(t2_hbm.at[idxc], b2, s2)

        @pl.loop(0, C // 16)
        def _grp(k):
            sl = pl.ds(k * 16, 16)
            ir = idxr[sl]
            ic = idxc[sl]
            bd0[sl] = plsc.load_gather(xt, [ir]) - plsc.load_gather(xt, [ic])
            ir1 = ir + XS
            ic1 = ic + XS
            bd1[sl] = plsc.load_gather(xt, [ir1]) - plsc.load_gather(xt, [ic1])
            ir2 = ir1 + XS
            ic2 = ic1 + XS
            bd2[sl] = plsc.load_gather(xt, [ir2]) - plsc.load_gather(xt, [ic2])

        cp1.wait()
        cp2.wait()
        pltpu.sync_copy(b1, br_hbm.at[pl.ds(base, C)])
        pltpu.sync_copy(b2, bc_hbm.at[pl.ds(base, C)])
        pltpu.sync_copy(bd0, d0_hbm.at[pl.ds(base, C)])
        pltpu.sync_copy(bd1, d1_hbm.at[pl.ds(base, C)])
        pltpu.sync_copy(bd2, d2_hbm.at[pl.ds(base, C)])


def _edge_body(br_ref, bc_ref, d0_ref, d1_ref, d2_ref, ea_ref,
               b3_ref, wd_ref, we_ref, we2_ref, be2_ref,
               wc1_ref, bc1_ref, wc2_ref,
               m_ref, t_ref):
    d0r = d0_ref[...].reshape(1, BE)
    d1r = d1_ref[...].reshape(1, BE)
    d2r = d2_ref[...].reshape(1, BE)
    dr3 = jnp.concatenate([d0r, d1r, d2r], axis=0)
    d16 = lax.dot_general(dr3, b3_ref[...], (((0,), (0,)), ((), ())),
                          precision=_HI)
    ds = jnp.sum(d16 * d16, axis=1, keepdims=True)
    pre = (br_ref[...] + bc_ref[...] + ds * wd_ref[...]
           + jnp.dot(ea_ref[...], we_ref[...], precision=_HI))
    m1 = jax.nn.silu(pre)
    m2 = jax.nn.silu(jnp.dot(m1, we2_ref[...]) + be2_ref[...])
    mc = jax.nn.silu(jnp.dot(m2, wc1_ref[...]) + bc1_ref[...])
    w = lax.dot_general(mc, wc2_ref[...], (((1,), (1,)), ((), ())),
                        precision=_HI)
    s = jnp.tanh(w) / (jnp.sqrt(ds + 1e-08) + 1e-08)
    m_ref[...] = m2
    t_ref[...] = jnp.concatenate(
        [d16 * s, jnp.zeros((BE, D - 16), jnp.float32)], axis=1)


def _edge_mlp(br, bc, d0, d1, d2, ea, b3, wd, we, we2, be2, wc1, bc1, wc2):
    return pl.pallas_call(
        _edge_body,
        grid=(E // BE,),
        in_specs=[
            pl.BlockSpec((BE, D), lambda i: (i, 0)),
            pl.BlockSpec((BE, D), lambda i: (i, 0)),
            pl.BlockSpec((BE,), lambda i: (i,)),
            pl.BlockSpec((BE,), lambda i: (i,)),
            pl.BlockSpec((BE,), lambda i: (i,)),
            pl.BlockSpec((BE, 16), lambda i: (i, 0)),
            pl.BlockSpec((3, 16), lambda i: (0, 0)),
            pl.BlockSpec((1, D), lambda i: (0, 0)),
            pl.BlockSpec((16, D), lambda i: (0, 0)),
            pl.BlockSpec((D, D), lambda i: (0, 0)),
            pl.BlockSpec((1, D), lambda i: (0, 0)),
            pl.BlockSpec((D, D), lambda i: (0, 0)),
            pl.BlockSpec((1, D), lambda i: (0, 0)),
            pl.BlockSpec((1, D), lambda i: (0, 0)),
        ],
        out_specs=[
            pl.BlockSpec((BE, D), lambda i: (i, 0)),
            pl.BlockSpec((BE, D), lambda i: (i, 0)),
        ],
        out_shape=[
            jax.ShapeDtypeStruct((E, D), jnp.float32),
            jax.ShapeDtypeStruct((E, D), jnp.float32),
        ],
    )(br, bc, d0, d1, d2, ea, b3, wd, we, we2, be2, wc1, bc1, wc2)


@functools.partial(
    pl.kernel,
    out_type=(
        jax.ShapeDtypeStruct((NC, NP, D), jnp.float32),
        jax.ShapeDtypeStruct((NC, NP, D), jnp.float32),
    ),
    mesh=_sc_mesh,
    compiler_params=pltpu.CompilerParams(needs_layout_passes=False),
    scratch_types=[
        pltpu.VMEM((C,), jnp.int32),
        pltpu.VMEM((C, D), jnp.float32),
        pltpu.VMEM_SHARED((NP, D), jnp.float32),
    ],
)
def _sc_scatter(m_hbm, t_hbm, row_hbm, zeros_hbm, pm_hbm, pt_hbm,
                idx, buf, acc):
    c = lax.axis_index("c")
    s = lax.axis_index("s")
    slab = pl.ds(s * RROWS, RROWS)
    base0 = (s * NC + c) * EW

    pltpu.sync_copy(zeros_hbm.at[slab], acc.at[slab])
    plsc.subcore_barrier()

    @pl.loop(0, NCH)
    def _mchunk(j):
        base = base0 + j * C
        pltpu.sync_copy(row_hbm.at[pl.ds(base, C)], idx)
        pltpu.sync_copy(m_hbm.at[pl.ds(base, C)], buf)
        pltpu.sync_copy(buf, acc.at[idx], add=True)

    plsc.subcore_barrier()
    pltpu.sync_copy(acc.at[slab], pm_hbm.at[c, slab])
    pltpu.sync_copy(zeros_hbm.at[slab], acc.at[slab])
    plsc.subcore_barrier()

    @pl.loop(0, NCH)
    def _tchunk(j):
        base = base0 + j * C
        pltpu.sync_copy(row_hbm.at[pl.ds(base, C)], idx)
        pltpu.sync_copy(t_hbm.at[pl.ds(base, C)], buf)
        pltpu.sync_copy(buf, acc.at[idx], add=True)

    plsc.subcore_barrier()
    pltpu.sync_copy(acc.at[slab], pt_hbm.at[c, slab])


def _node_body(h_ref, xp_ref, pm_ref, pt_ref,
               wn1h_ref, wn1m_ref, bn1_ref, wn2_ref, bn2_ref,
               hn_ref, xn_ref):
    hb = h_ref[...]
    aggm = pm_ref[0] + pm_ref[1]
    aggt = pt_ref[0] + pt_ref[1]
    u = jax.nn.silu(jnp.dot(hb, wn1h_ref[...], precision=_HI)
                    + jnp.dot(aggm, wn1m_ref[...], precision=_HI)
                    + bn1_ref[...])
    hn_ref[...] = hb + jnp.dot(u, wn2_ref[...], precision=_HI) + bn2_ref[...]
    xn_ref[...] = xp_ref[...] + aggt[:, :16]


def _node_mlp(h, xp, pm, pt, wn1h, wn1m, bn1, wn2, bn2):
    return pl.pallas_call(
        _node_body,
        grid=(N // BN,),
        in_specs=[
            pl.BlockSpec((BN, D), lambda i: (i, 0)),
            pl.BlockSpec((BN, 16), lambda i: (i, 0)),
            pl.BlockSpec((NC, BN, D), lambda i: (0, i, 0)),
            pl.BlockSpec((NC, BN, D), lambda i: (0, i, 0)),
            pl.BlockSpec((D, D), lambda i: (0, 0)),
            pl.BlockSpec((D, D), lambda i: (0, 0)),
            pl.BlockSpec((1, D), lambda i: (0, 0)),
            pl.BlockSpec((D, D), lambda i: (0, 0)),
            pl.BlockSpec((1, D), lambda i: (0, 0)),
        ],
        out_specs=[
            pl.BlockSpec((BN, D), lambda i: (i, 0)),
            pl.BlockSpec((BN, 16), lambda i: (i, 0)),
        ],
        out_shape=[
            jax.ShapeDtypeStruct((N, D), jnp.float32),
            jax.ShapeDtypeStruct((N, 16), jnp.float32),
        ],
    )(h, xp, pm, pt, wn1h, wn1m, bn1, wn2, bn2)


def kernel(h, x, edge_index, edge_attr,
           We1, be1, We2, be2, Wc1, bc1, Wc2, Wn1, bn1, Wn2, bn2):
    row = edge_index[0].astype(jnp.int32)
    col = edge_index[1].astype(jnp.int32)
    xf = jnp.pad(x.T, ((0, 0), (0, XS - N))).reshape(-1)
    xp = jnp.pad(x, ((0, 0), (0, 13)))

    wa = We1[:D]
    wb = We1[D:2 * D]
    wd = We1[2 * D:2 * D + 1]
    we = We1[2 * D + 1:]
    b3 = jnp.eye(3, 16, dtype=jnp.float32)

    t1, t2 = _node_pre(h, wa, wb, be1.reshape(1, D))
    br, bc, d0, d1, d2 = _sc_gather(t1, t2, xf, row, col)
    m_ij, tt = _edge_mlp(br, bc, d0, d1, d2, edge_attr, b3,
                         wd, we, We2, be2.reshape(1, D),
                         Wc1, bc1.reshape(1, D), Wc2.reshape(1, D))

    zeros = jnp.zeros((NP, D), jnp.float32)
    pm, pt = _sc_scatter(m_ij, tt, row, zeros)

    hn, xn = _node_mlp(h, xp, pm, pt,
                       Wn1[:D], Wn1[D:], bn1.reshape(1, D),
                       Wn2, bn2.reshape(1, D))
    return hn, xn[:, :3]


# speedup vs baseline: 3.0000x; 1.1083x over previous
import functools

import jax
import jax.numpy as jnp
from jax import lax
from jax.experimental import pallas as pl
from jax.experimental.pallas import tpu as pltpu
from jax.experimental.pallas import tpu_sc as plsc

N = 10000
E = 320000
EH = E // 2
D = 128
NC = 2
NS = 16
NW = NC * NS
EW = EH // NW
C = 40
NCH = EW // C
NP = 10112
RROWS = NP // NS
XS = 10240

BN = 1000
BE = 640

_HI = lax.Precision.HIGHEST

_sc_mesh = plsc.VectorSubcoreMesh(core_axis_name="c", subcore_axis_name="s")


def _pre_body(h_ref, wa_ref, wb_ref, be1_ref, t1_ref, t2_ref):
    hb = h_ref[...]
    t1_ref[...] = jnp.dot(hb, wa_ref[...], precision=_HI) + be1_ref[...]
    t2_ref[...] = jnp.dot(hb, wb_ref[...], precision=_HI)


def _node_pre(h, wa, wb, be1):
    return pl.pallas_call(
        _pre_body,
        grid=(N // BN,),
        in_specs=[
            pl.BlockSpec((BN, D), lambda i: (i, 0)),
            pl.BlockSpec((D, D), lambda i: (0, 0)),
            pl.BlockSpec((D, D), lambda i: (0, 0)),
            pl.BlockSpec((1, D), lambda i: (0, 0)),
        ],
        out_specs=[
            pl.BlockSpec((BN, D), lambda i: (i, 0)),
            pl.BlockSpec((BN, D), lambda i: (i, 0)),
        ],
        out_shape=[
            jax.ShapeDtypeStruct((N, D), jnp.float32),
            jax.ShapeDtypeStruct((N, D), jnp.float32),
        ],
    )(h, wa, wb, be1)


@functools.partial(
    pl.kernel,
    out_type=(
        jax.ShapeDtypeStruct((EH, D), jnp.float32),
        jax.ShapeDtypeStruct((EH, D), jnp.float32),
        jax.ShapeDtypeStruct((EH,), jnp.float32),
        jax.ShapeDtypeStruct((EH,), jnp.float32),
        jax.ShapeDtypeStruct((EH,), jnp.float32),
    ),
    mesh=_sc_mesh,
    compiler_params=pltpu.CompilerParams(needs_layout_passes=False),
    scratch_types=[
        pltpu.VMEM((C,), jnp.int32),
        pltpu.VMEM((C,), jnp.int32),
        pltpu.VMEM((C, D), jnp.float32),
        pltpu.VMEM((C, D), jnp.float32),
        pltpu.VMEM((3 * XS,), jnp.float32),
        pltpu.VMEM((C,), jnp.float32),
        pltpu.VMEM((C,), jnp.float32),
        pltpu.VMEM((C,), jnp.float32),
        pltpu.SemaphoreType.DMA,
        pltpu.SemaphoreType.DMA,
    ],
)
def _sc_gather(t1_hbm, t2_hbm, xf_hbm, row_hbm, col_hbm,
               br_hbm, bc_hbm, d0_hbm, d1_hbm, d2_hbm,
               idxr, idxc, b1, b2, xt, bd0, bd1, bd2, s1, s2):
    wid = lax.axis_index("s") * NC + lax.axis_index("c")
    base0 = wid * EW

    pltpu.sync_copy(xf_hbm, xt)

    @pl.loop(0, NCH)
    def _chunk(j):
        base = base0 + j * C
        pltpu.sync_copy(row_hbm.at[pl.ds(base, C)], idxr)
        pltpu.sync_copy(col_hbm.at[pl.ds(base, C)], idxc)
        cp1 = pltpu.async_copy(t1_hbm.at[idxr], b1, s1)
        cp2 = pltpu.async_copy(t2_hbm.at[idxc], b2, s2)

        @pl.loop(0, C // 16)
        def _grp(k):
            sl = pl.ds(k * 16, 16)
            ir = idxr[sl]
            ic = idxc[sl]
            bd0[sl] = plsc.load_gather(xt, [ir]) - plsc.load_gather(xt, [ic])
            ir1 = ir + XS
            ic1 = ic + XS
            bd1[sl] = plsc.load_gather(xt, [ir1]) - plsc.load_gather(xt, [ic1])
            ir2 = ir1 + XS
            ic2 = ic1 + XS
            bd2[sl] = plsc.load_gather(xt, [ir2]) - plsc.load_gather(xt, [ic2])

        cp1.wait()
        cp2.wait()
        pltpu.sync_copy(b1, br_hbm.at[pl.ds(base, C)])
        pltpu.sync_copy(b2, bc_hbm.at[pl.ds(base, C)])
        pltpu.sync_copy(bd0, d0_hbm.at[pl.ds(base, C)])
        pltpu.sync_copy(bd1, d1_hbm.at[pl.ds(base, C)])
        pltpu.sync_copy(bd2, d2_hbm.at[pl.ds(base, C)])


def _edge_body(br_ref, bc_ref, d0_ref, d1_ref, d2_ref, ea_ref,
               b3_ref, wd_ref, we_ref, we2_ref, be2_ref,
               wc1_ref, bc1_ref, wc2_ref,
               m_ref, t_ref):
    d0r = d0_ref[...].reshape(1, BE)
    d1r = d1_ref[...].reshape(1, BE)
    d2r = d2_ref[...].reshape(1, BE)
    dr3 = jnp.concatenate([d0r, d1r, d2r], axis=0)
    d16 = lax.dot_general(dr3, b3_ref[...], (((0,), (0,)), ((), ())),
                          precision=_HI)
    ds = jnp.sum(d16 * d16, axis=1, keepdims=True)
    pre = (br_ref[...] + bc_ref[...] + ds * wd_ref[...]
           + jnp.dot(ea_ref[...], we_ref[...], precision=_HI))
    m1 = jax.nn.silu(pre)
    m2 = jax.nn.silu(jnp.dot(m1, we2_ref[...]) + be2_ref[...])
    mc = jax.nn.silu(jnp.dot(m2, wc1_ref[...]) + bc1_ref[...])
    w = lax.dot_general(mc, wc2_ref[...], (((1,), (1,)), ((), ())),
                        precision=_HI)
    s = jnp.tanh(w) / (jnp.sqrt(ds + 1e-08) + 1e-08)
    m_ref[...] = m2
    t_ref[...] = jnp.concatenate(
        [d16 * s, jnp.zeros((BE, D - 16), jnp.float32)], axis=1)


def _edge_mlp(br, bc, d0, d1, d2, ea, b3, wd, we, we2, be2, wc1, bc1, wc2):
    d0 = d0.reshape(EH // BE, 1, BE)
    d1 = d1.reshape(EH // BE, 1, BE)
    d2 = d2.reshape(EH // BE, 1, BE)
    return pl.pallas_call(
        _edge_body,
        grid=(EH // BE,),
        in_specs=[
            pl.BlockSpec((BE, D), lambda i: (i, 0)),
            pl.BlockSpec((BE, D), lambda i: (i, 0)),
            pl.BlockSpec((1, 1, BE), lambda i: (i, 0, 0)),
            pl.BlockSpec((1, 1, BE), lambda i: (i, 0, 0)),
            pl.BlockSpec((1, 1, BE), lambda i: (i, 0, 0)),
            pl.BlockSpec((BE, 16), lambda i: (i, 0)),
            pl.BlockSpec((3, 16), lambda i: (0, 0)),
            pl.BlockSpec((1, D), lambda i: (0, 0)),
            pl.BlockSpec((16, D), lambda i: (0, 0)),
            pl.BlockSpec((D, D), lambda i: (0, 0)),
            pl.BlockSpec((1, D), lambda i: (0, 0)),
            pl.BlockSpec((D, D), lambda i: (0, 0)),
            pl.BlockSpec((1, D), lambda i: (0, 0)),
            pl.BlockSpec((1, D), lambda i: (0, 0)),
        ],
        out_specs=[
            pl.BlockSpec((BE, D), lambda i: (i, 0)),
            pl.BlockSpec((BE, D), lambda i: (i, 0)),
        ],
        out_shape=[
            jax.ShapeDtypeStruct((EH, D), jnp.float32),
            jax.ShapeDtypeStruct((EH, D), jnp.float32),
        ],
    )(br, bc, d0, d1, d2, ea, b3, wd, we, we2, be2, wc1, bc1, wc2)


@functools.partial(
    pl.kernel,
    out_type=(
        jax.ShapeDtypeStruct((NC, NP, D), jnp.float32),
        jax.ShapeDtypeStruct((NC, NP, D), jnp.float32),
    ),
    mesh=_sc_mesh,
    compiler_params=pltpu.CompilerParams(needs_layout_passes=False),
    scratch_types=[
        pltpu.VMEM((C,), jnp.int32),
        pltpu.VMEM((C, D), jnp.float32),
        pltpu.VMEM_SHARED((NP, D), jnp.float32),
    ],
)
def _sc_scatter(m_hbm, t_hbm, row_hbm, im_hbm, it_hbm, pm_hbm, pt_hbm,
                idx, buf, acc):
    c = lax.axis_index("c")
    s = lax.axis_index("s")
    slab = pl.ds(s * RROWS, RROWS)
    base0 = (s * NC + c) * EW

    pltpu.sync_copy(im_hbm.at[c, slab], acc.at[slab])
    plsc.subcore_barrier()

    @pl.loop(0, NCH)
    def _mchunk(j):
        base = base0 + j * C
        pltpu.sync_copy(row_hbm.at[pl.ds(base, C)], idx)
        pltpu.sync_copy(m_hbm.at[pl.ds(base, C)], buf)
        pltpu.sync_copy(buf, acc.at[idx], add=True)

    plsc.subcore_barrier()
    pltpu.sync_copy(acc.at[slab], pm_hbm.at[c, slab])
    pltpu.sync_copy(it_hbm.at[c, slab], acc.at[slab])
    plsc.subcore_barrier()

    @pl.loop(0, NCH)
    def _tchunk(j):
        base = base0 + j * C
        pltpu.sync_copy(row_hbm.at[pl.ds(base, C)], idx)
        pltpu.sync_copy(t_hbm.at[pl.ds(base, C)], buf)
        pltpu.sync_copy(buf, acc.at[idx], add=True)

    plsc.subcore_barrier()
    pltpu.sync_copy(acc.at[slab], pt_hbm.at[c, slab])


def _node_body(h_ref, xp_ref, pm_ref, pt_ref,
               wn1h_ref, wn1m_ref, bn1_ref, wn2_ref, bn2_ref,
               hn_ref, xn_ref):
    hb = h_ref[...]
    aggm = pm_ref[0] + pm_ref[1]
    aggt = pt_ref[0] + pt_ref[1]
    u = jax.nn.silu(jnp.dot(hb, wn1h_ref[...], precision=_HI)
                    + jnp.dot(aggm, wn1m_ref[...], precision=_HI)
                    + bn1_ref[...])
    hn_ref[...] = hb + jnp.dot(u, wn2_ref[...], precision=_HI) + bn2_ref[...]
    xn_ref[...] = xp_ref[...] + aggt[:, :16]


def _node_mlp(h, xp, pm, pt, wn1h, wn1m, bn1, wn2, bn2):
    return pl.pallas_call(
        _node_body,
        grid=(N // BN,),
        in_specs=[
            pl.BlockSpec((BN, D), lambda i: (i, 0)),
            pl.BlockSpec((BN, 16), lambda i: (i, 0)),
            pl.BlockSpec((NC, BN, D), lambda i: (0, i, 0)),
            pl.BlockSpec((NC, BN, D), lambda i: (0, i, 0)),
            pl.BlockSpec((D, D), lambda i: (0, 0)),
            pl.BlockSpec((D, D), lambda i: (0, 0)),
            pl.BlockSpec((1, D), lambda i: (0, 0)),
            pl.BlockSpec((D, D), lambda i: (0, 0)),
            pl.BlockSpec((1, D), lambda i: (0, 0)),
        ],
        out_specs=[
            pl.BlockSpec((BN, D), lambda i: (i, 0)),
            pl.BlockSpec((BN, 16), lambda i: (i, 0)),
        ],
        out_shape=[
            jax.ShapeDtypeStruct((N, D), jnp.float32),
            jax.ShapeDtypeStruct((N, 16), jnp.float32),
        ],
    )(h, xp, pm, pt, wn1h, wn1m, bn1, wn2, bn2)


def kernel(h, x, edge_index, edge_attr,
           We1, be1, We2, be2, Wc1, bc1, Wc2, Wn1, bn1, Wn2, bn2):
    row = edge_index[0].astype(jnp.int32)
    col = edge_index[1].astype(jnp.int32)
    xf = jnp.pad(x.T, ((0, 0), (0, XS - N))).reshape(-1)
    xp = jnp.pad(x, ((0, 0), (0, 13)))

    wa = We1[:D]
    wb = We1[D:2 * D]
    wd = We1[2 * D:2 * D + 1]
    we = We1[2 * D + 1:]
    b3 = jnp.eye(3, 16, dtype=jnp.float32)
    be2r = be2.reshape(1, D)
    bc1r = bc1.reshape(1, D)
    wc2r = Wc2.reshape(1, D)

    t1, t2 = _node_pre(h, wa, wb, be1.reshape(1, D))

    pm = jnp.zeros((NC, NP, D), jnp.float32)
    pt = jnp.zeros((NC, NP, D), jnp.float32)
    halves = []
    for k in range(2):
        sl = slice(k * EH, (k + 1) * EH)
        halves.append(_sc_gather(t1, t2, xf, row[sl], col[sl]))
    for k in range(2):
        br, bc, d0, d1, d2 = halves[k]
        sl = slice(k * EH, (k + 1) * EH)
        m_ij, tt = _edge_mlp(br, bc, d0, d1, d2, edge_attr[sl], b3,
                             wd, we, We2, be2r, Wc1, bc1r, wc2r)
        pm, pt = _sc_scatter(m_ij, tt, row[sl], pm, pt)

    hn, xn = _node_mlp(h, xp, pm, pt,
                       Wn1[:D], Wn1[D:], bn1.reshape(1, D),
                       Wn2, bn2.reshape(1, D))
    return hn, xn[:, :3]
